# baseline (speedup 1.0000x reference)
import functools

import jax
import jax.numpy as jnp
from jax import lax
from jax.experimental import pallas as pl
from jax.experimental.pallas import tpu as pltpu
from jax.experimental.pallas import tpu_sc as plsc

N = 10000
NPAD = 10240
E = 320000
IN_DIM = 128
HID = 64
OUT = 32

NC = 2
NS = 16
NW = NC * NS
CH = 128
K = 80
EPT = K * CH
EPAD = EPT * NW
RPT = NPAD // NS
Q = 8
F = 4
CHA = 64
KA = EPT // CHA
QA = 5
FA = 2

def _mesh():
    return plsc.VectorSubcoreMesh(
        core_axis_name="c", subcore_axis_name="s",
        num_cores=NC, num_subcores=NS)


def _deg_kernel():
    @functools.partial(
        pl.kernel,
        out_type=jax.ShapeDtypeStruct((NC, NPAD), jnp.float32),
        mesh=_mesh(),
        compiler_params=pltpu.CompilerParams(use_tc_tiling_on_sc=False),
        scratch_types=[
            pltpu.VMEM((K, CH), jnp.int32),
            pltpu.VMEM((CH,), jnp.float32),
            pltpu.VMEM((RPT,), jnp.float32),
            pltpu.VMEM_SHARED((NPAD,), jnp.float32),
        ],
    )
    def deg(col_hbm, out_hbm, col_v, ones_v, stage_v, acc_sh):
        c = lax.axis_index("c")
        s = lax.axis_index("s")
        eb = c * NS + s
        r0 = s * RPT

        def fill_ones(i, carry):
            ones_v[pl.ds(i * 16, 16)] = jnp.full((16,), 1.0, jnp.float32)
            return carry

        lax.fori_loop(0, CH // 16, fill_ones, 0)

        def fill_zero(i, carry):
            stage_v[pl.ds(i * 16, 16)] = jnp.zeros((16,), jnp.float32)
            return carry

        lax.fori_loop(0, RPT // 16, fill_zero, 0)
        pltpu.sync_copy(stage_v, acc_sh.at[pl.ds(r0, RPT)])
        pltpu.sync_copy(col_hbm.at[eb], col_v)
        plsc.subcore_barrier()

        def body(j, carry):
            pltpu.sync_copy(ones_v, acc_sh.at[col_v.at[j]], add=True)
            return carry

        lax.fori_loop(0, K, body, 0)
        plsc.subcore_barrier()
        pltpu.sync_copy(acc_sh.at[pl.ds(r0, RPT)], stage_v)
        pltpu.sync_copy(stage_v, out_hbm.at[c, pl.ds(r0, RPT)])

    return deg


def _agg_kernel(D):
    @functools.partial(
        pl.kernel,
        out_type=jax.ShapeDtypeStruct((NC, NPAD, D), jnp.float32),
        mesh=_mesh(),
        compiler_params=pltpu.CompilerParams(use_tc_tiling_on_sc=False),
        scratch_types=[
            pltpu.VMEM((KA, CHA), jnp.int32),
            pltpu.VMEM((KA, CHA), jnp.int32),
            pltpu.VMEM((QA, CHA, D), jnp.float32),
            pltpu.SemaphoreType.DMA((QA,)),
            pltpu.SemaphoreType.DMA((QA,)),
            pltpu.VMEM_SHARED((NPAD, D), jnp.float32),
            pltpu.VMEM_SHARED((NPAD, D), jnp.float32),
        ],
    )
    def agg(row_hbm, col_hbm, g_hbm, out_hbm,
            row_v, col_v, bufs, gsem, ssem, g_sh, acc_sh):
        c = lax.axis_index("c")
        s = lax.axis_index("s")
        eb = c * NS + s
        r0 = s * RPT

        def stage_in(t, carry):
            pltpu.sync_copy(g_hbm.at[pl.ds(r0 + t * CHA, CHA)], bufs.at[0])
            pltpu.sync_copy(bufs.at[0], g_sh.at[pl.ds(r0 + t * CHA, CHA)])
            pltpu.sync_copy(bufs.at[0], acc_sh.at[pl.ds(r0 + t * CHA, CHA)])
            return carry

        lax.fori_loop(0, RPT // CHA, stage_in, 0)
        pltpu.sync_copy(row_hbm.at[eb], row_v)
        pltpu.sync_copy(col_hbm.at[eb], col_v)
        plsc.subcore_barrier()

        for b in range(FA):
            pltpu.async_copy(g_sh.at[row_v.at[b]], bufs.at[b], gsem.at[b])

        def outer(jo, carry):
            for b in range(QA):
                j = jo * QA + b
                pltpu.make_async_copy(
                    g_sh.at[row_v.at[j]], bufs.at[b], gsem.at[b]).wait()
                pltpu.async_copy(
                    bufs.at[b], acc_sh.at[col_v.at[j]], ssem.at[b], add=True)
                jf = j + FA
                bf = (b + FA) % QA

                @pl.when(jf < KA)
                def _fire():
                    @pl.when(jf >= QA)
                    def _drain():
                        pltpu.make_async_copy(
                            g_hbm.at[pl.ds(0, CHA)], bufs.at[bf],
                            ssem.at[bf]).wait()

                    pltpu.async_copy(
                        g_sh.at[row_v.at[jf]], bufs.at[bf], gsem.at[bf])
            return carry

        lax.fori_loop(0, KA // QA, outer, 0)
        for b in range(QA):
            pltpu.make_async_copy(
                g_hbm.at[pl.ds(0, CHA)], bufs.at[b], ssem.at[b]).wait()
        plsc.subcore_barrier()

        def stage_out(t, carry):
            pltpu.sync_copy(acc_sh.at[pl.ds(r0 + t * CHA, CHA)], bufs.at[0])
            pltpu.sync_copy(bufs.at[0], out_hbm.at[c, pl.ds(r0 + t * CHA, CHA)])
            return carry

        lax.fori_loop(0, RPT // CHA, stage_out, 0)

    return agg


_R = 1024


def _dinv_block(dp_ref):
    deg = dp_ref[:, 0:1] + dp_ref[:, 1:2] + 1.0
    return lax.rsqrt(deg)


def _tc_a(x_pad, W1, dpT):
    def body(x_ref, w_ref, dp_ref, o_ref):
        dinv = _dinv_block(dp_ref)
        o_ref[...] = jnp.dot(x_ref[...], w_ref[...],
                             preferred_element_type=jnp.float32) * dinv

    return pl.pallas_call(
        body,
        grid=(NPAD // _R,),
        in_specs=[
            pl.BlockSpec((_R, IN_DIM), lambda i: (i, 0)),
            pl.BlockSpec((IN_DIM, HID), lambda i: (0, 0)),
            pl.BlockSpec((_R, NC), lambda i: (i, 0)),
        ],
        out_specs=pl.BlockSpec((_R, HID), lambda i: (i, 0)),
        out_shape=jax.ShapeDtypeStruct((NPAD, HID), jnp.float32),
    )(x_pad, W1, dpT)


def _tc_b(p0, p1, g1, dpT, b1, W2):
    def body(p0_ref, p1_ref, g1_ref, dp_ref, b_ref, w_ref, o_ref):
        dinv = _dinv_block(dp_ref)
        pre = (p0_ref[...] + p1_ref[...] - g1_ref[...]) * dinv + b_ref[...]
        h = jnp.maximum(pre, 0.0)
        o_ref[...] = jnp.dot(h, w_ref[...],
                             preferred_element_type=jnp.float32) * dinv

    return pl.pallas_call(
        body,
        grid=(NPAD // _R,),
        in_specs=[
            pl.BlockSpec((_R, HID), lambda i: (i, 0)),
            pl.BlockSpec((_R, HID), lambda i: (i, 0)),
            pl.BlockSpec((_R, HID), lambda i: (i, 0)),
            pl.BlockSpec((_R, NC), lambda i: (i, 0)),
            pl.BlockSpec((1, HID), lambda i: (0, 0)),
            pl.BlockSpec((HID, OUT), lambda i: (0, 0)),
        ],
        out_specs=pl.BlockSpec((_R, OUT), lambda i: (i, 0)),
        out_shape=jax.ShapeDtypeStruct((NPAD, OUT), jnp.float32),
    )(p0, p1, g1, dpT, b1, W2)


def _tc_c(p0, p1, g2, dpT, b2):
    def body(p0_ref, p1_ref, g2_ref, dp_ref, b_ref, o_ref):
        dinv = _dinv_block(dp_ref)
        o_ref[...] = (p0_ref[...] + p1_ref[...] - g2_ref[...]) * dinv + b_ref[...]

    return pl.pallas_call(
        body,
        grid=(NPAD // _R,),
        in_specs=[
            pl.BlockSpec((_R, OUT), lambda i: (i, 0)),
            pl.BlockSpec((_R, OUT), lambda i: (i, 0)),
            pl.BlockSpec((_R, OUT), lambda i: (i, 0)),
            pl.BlockSpec((_R, NC), lambda i: (i, 0)),
            pl.BlockSpec((1, OUT), lambda i: (0, 0)),
        ],
        out_specs=pl.BlockSpec((_R, OUT), lambda i: (i, 0)),
        out_shape=jax.ShapeDtypeStruct((NPAD, OUT), jnp.float32),
    )(p0, p1, g2, dpT, b2)


def kernel(x, edge_index, W1, b1, W2, b2):
    row = edge_index[0].astype(jnp.int32)
    col = edge_index[1].astype(jnp.int32)
    npad_extra = NPAD - N
    epad = EPAD - E
    pad_i = jnp.arange(epad, dtype=jnp.int32)
    row_p = jnp.concatenate([row, (pad_i * 97) % N]).reshape(NW, K, CH)
    col_p = jnp.concatenate([col, N + pad_i % npad_extra]).reshape(NW, K, CH)
    x_pad = jnp.concatenate(
        [x, jnp.zeros((npad_extra, IN_DIM), x.dtype)])

    deg_part = _deg_kernel()(col_p)
    dpT = deg_part.T
    g1 = _tc_a(x_pad, W1, dpT)
    row_a = row_p.reshape(NW, KA, CHA)
    col_a = col_p.reshape(NW, KA, CHA)
    agg1 = _agg_kernel(HID)(row_a, col_a, g1)
    g2 = _tc_b(agg1[0], agg1[1], g1, dpT, b1.reshape(1, HID), W2)
    agg2 = _agg_kernel(OUT)(row_a, col_a, g2)
    z_pad = _tc_c(agg2[0], agg2[1], g2, dpT, b2.reshape(1, OUT))
    return z_pad[:N]


# speedup vs baseline: 1.0877x; 1.0877x over previous
import functools

import jax
import jax.numpy as jnp
from jax import lax
from jax.experimental import pallas as pl
from jax.experimental.pallas import tpu as pltpu
from jax.experimental.pallas import tpu_sc as plsc

N = 10000
NPAD = 10240
E = 320000
IN_DIM = 128
HID = 64
OUT = 32

NC = 2
NS = 16
NW = NC * NS
CH = 128
K = 80
EPT = K * CH
EPAD = EPT * NW
RPT = NPAD // NS
Q = 8
F = 4

def _mesh():
    return plsc.VectorSubcoreMesh(
        core_axis_name="c", subcore_axis_name="s",
        num_cores=NC, num_subcores=NS)


def _deg_kernel():
    @functools.partial(
        pl.kernel,
        out_type=jax.ShapeDtypeStruct((NC, NPAD), jnp.float32),
        mesh=_mesh(),
        compiler_params=pltpu.CompilerParams(use_tc_tiling_on_sc=False),
        scratch_types=[
            pltpu.VMEM((K, CH), jnp.int32),
            pltpu.VMEM((CH,), jnp.float32),
            pltpu.VMEM((RPT,), jnp.float32),
            pltpu.VMEM_SHARED((NPAD,), jnp.float32),
        ],
    )
    def deg(col_hbm, out_hbm, col_v, ones_v, stage_v, acc_sh):
        c = lax.axis_index("c")
        s = lax.axis_index("s")
        eb = c * NS + s
        r0 = s * RPT

        def fill_ones(i, carry):
            ones_v[pl.ds(i * 16, 16)] = jnp.full((16,), 1.0, jnp.float32)
            return carry

        lax.fori_loop(0, CH // 16, fill_ones, 0)

        def fill_zero(i, carry):
            stage_v[pl.ds(i * 16, 16)] = jnp.zeros((16,), jnp.float32)
            return carry

        lax.fori_loop(0, RPT // 16, fill_zero, 0)
        pltpu.sync_copy(stage_v, acc_sh.at[pl.ds(r0, RPT)])
        pltpu.sync_copy(col_hbm.at[eb], col_v)
        plsc.subcore_barrier()

        def body(j, carry):
            pltpu.sync_copy(ones_v, acc_sh.at[col_v.at[j]], add=True)
            return carry

        lax.fori_loop(0, K, body, 0)
        plsc.subcore_barrier()
        pltpu.sync_copy(acc_sh.at[pl.ds(r0, RPT)], stage_v)
        pltpu.sync_copy(stage_v, out_hbm.at[c, pl.ds(r0, RPT)])

    return deg


def _agg_kernel(D, spmem_gather):
    scratch = [
        pltpu.VMEM((K, CH), jnp.int32),
        pltpu.VMEM((K, CH), jnp.int32),
        pltpu.VMEM((Q, CH, D), jnp.float32),
        pltpu.SemaphoreType.DMA((Q,)),
        pltpu.SemaphoreType.DMA((Q,)),
        pltpu.VMEM_SHARED((NPAD, D), jnp.float32),
    ]
    if spmem_gather:
        scratch.append(pltpu.VMEM_SHARED((NPAD, D), jnp.float32))

    @functools.partial(
        pl.kernel,
        out_type=jax.ShapeDtypeStruct((NC, NPAD, D), jnp.float32),
        mesh=_mesh(),
        compiler_params=pltpu.CompilerParams(use_tc_tiling_on_sc=False),
        scratch_types=scratch,
    )
    def agg(row_hbm, col_hbm, g_hbm, out_hbm,
            row_v, col_v, bufs, gsem, ssem, acc_sh, *maybe_gsh):
        c = lax.axis_index("c")
        s = lax.axis_index("s")
        eb = c * NS + s
        r0 = s * RPT
        g_src = maybe_gsh[0] if spmem_gather else g_hbm

        def stage_in(t, carry):
            pltpu.sync_copy(g_hbm.at[pl.ds(r0 + t * CH, CH)], bufs.at[0])
            pltpu.sync_copy(bufs.at[0], acc_sh.at[pl.ds(r0 + t * CH, CH)])
            if spmem_gather:
                pltpu.sync_copy(bufs.at[0], maybe_gsh[0].at[pl.ds(r0 + t * CH, CH)])
            return carry

        lax.fori_loop(0, RPT // CH, stage_in, 0)
        pltpu.sync_copy(row_hbm.at[eb], row_v)
        pltpu.sync_copy(col_hbm.at[eb], col_v)
        plsc.subcore_barrier()

        for b in range(F):
            pltpu.async_copy(g_src.at[row_v.at[b]], bufs.at[b], gsem.at[b])

        def outer(jo, carry):
            for b in range(Q):
                j = jo * Q + b
                pltpu.make_async_copy(
                    g_src.at[row_v.at[j]], bufs.at[b], gsem.at[b]).wait()
                pltpu.async_copy(
                    bufs.at[b], acc_sh.at[col_v.at[j]], ssem.at[b], add=True)
                jf = j + F
                bf = (b + F) % Q

                @pl.when(jf < K)
                def _fire():
                    @pl.when(jf >= Q)
                    def _drain():
                        pltpu.make_async_copy(
                            g_hbm.at[pl.ds(0, CH)], bufs.at[bf],
                            ssem.at[bf]).wait()

                    pltpu.async_copy(
                        g_src.at[row_v.at[jf]], bufs.at[bf], gsem.at[bf])
            return carry

        lax.fori_loop(0, K // Q, outer, 0)
        for b in range(Q):
            pltpu.make_async_copy(
                g_hbm.at[pl.ds(0, CH)], bufs.at[b], ssem.at[b]).wait()
        plsc.subcore_barrier()

        def stage_out(t, carry):
            pltpu.sync_copy(acc_sh.at[pl.ds(r0 + t * CH, CH)], bufs.at[0])
            pltpu.sync_copy(bufs.at[0], out_hbm.at[c, pl.ds(r0 + t * CH, CH)])
            return carry

        lax.fori_loop(0, RPT // CH, stage_out, 0)

    return agg


_R = 1024


def _dinv_block(dp_ref):
    deg = dp_ref[:, 0:1] + dp_ref[:, 1:2] + 1.0
    return lax.rsqrt(deg)


def _tc_a(x_pad, W1, dpT):
    def body(x_ref, w_ref, dp_ref, o_ref):
        dinv = _dinv_block(dp_ref)
        o_ref[...] = jnp.dot(x_ref[...], w_ref[...],
                             preferred_element_type=jnp.float32) * dinv

    return pl.pallas_call(
        body,
        grid=(NPAD // _R,),
        in_specs=[
            pl.BlockSpec((_R, IN_DIM), lambda i: (i, 0)),
            pl.BlockSpec((IN_DIM, HID), lambda i: (0, 0)),
            pl.BlockSpec((_R, NC), lambda i: (i, 0)),
        ],
        out_specs=pl.BlockSpec((_R, HID), lambda i: (i, 0)),
        out_shape=jax.ShapeDtypeStruct((NPAD, HID), jnp.float32),
    )(x_pad, W1, dpT)


def _tc_b(p0, p1, g1, dpT, b1, W2):
    def body(p0_ref, p1_ref, g1_ref, dp_ref, b_ref, w_ref, o_ref):
        dinv = _dinv_block(dp_ref)
        pre = (p0_ref[...] + p1_ref[...] - g1_ref[...]) * dinv + b_ref[...]
        h = jnp.maximum(pre, 0.0)
        o_ref[...] = jnp.dot(h, w_ref[...],
                             preferred_element_type=jnp.float32) * dinv

    return pl.pallas_call(
        body,
        grid=(NPAD // _R,),
        in_specs=[
            pl.BlockSpec((_R, HID), lambda i: (i, 0)),
            pl.BlockSpec((_R, HID), lambda i: (i, 0)),
            pl.BlockSpec((_R, HID), lambda i: (i, 0)),
            pl.BlockSpec((_R, NC), lambda i: (i, 0)),
            pl.BlockSpec((1, HID), lambda i: (0, 0)),
            pl.BlockSpec((HID, OUT), lambda i: (0, 0)),
        ],
        out_specs=pl.BlockSpec((_R, OUT), lambda i: (i, 0)),
        out_shape=jax.ShapeDtypeStruct((NPAD, OUT), jnp.float32),
    )(p0, p1, g1, dpT, b1, W2)


def _tc_c(p0, p1, g2, dpT, b2):
    def body(p0_ref, p1_ref, g2_ref, dp_ref, b_ref, o_ref):
        dinv = _dinv_block(dp_ref)
        o_ref[...] = (p0_ref[...] + p1_ref[...] - g2_ref[...]) * dinv + b_ref[...]

    return pl.pallas_call(
        body,
        grid=(NPAD // _R,),
        in_specs=[
            pl.BlockSpec((_R, OUT), lambda i: (i, 0)),
            pl.BlockSpec((_R, OUT), lambda i: (i, 0)),
            pl.BlockSpec((_R, OUT), lambda i: (i, 0)),
            pl.BlockSpec((_R, NC), lambda i: (i, 0)),
            pl.BlockSpec((1, OUT), lambda i: (0, 0)),
        ],
        out_specs=pl.BlockSpec((_R, OUT), lambda i: (i, 0)),
        out_shape=jax.ShapeDtypeStruct((NPAD, OUT), jnp.float32),
    )(p0, p1, g2, dpT, b2)


def kernel(x, edge_index, W1, b1, W2, b2):
    row = edge_index[0].astype(jnp.int32)
    col = edge_index[1].astype(jnp.int32)
    npad_extra = NPAD - N
    epad = EPAD - E
    pad_i = jnp.arange(epad, dtype=jnp.int32)
    row_p = jnp.concatenate([row, (pad_i * 97) % N]).reshape(NW, K, CH)
    col_p = jnp.concatenate([col, N + pad_i % npad_extra]).reshape(NW, K, CH)
    x_pad = jnp.concatenate(
        [x, jnp.zeros((npad_extra, IN_DIM), x.dtype)])

    deg_part = _deg_kernel()(col_p)
    dpT = deg_part.T
    g1 = _tc_a(x_pad, W1, dpT)
    agg1 = _agg_kernel(HID, False)(row_p, col_p, g1)
    g2 = _tc_b(agg1[0], agg1[1], g1, dpT, b1.reshape(1, HID), W2)
    agg2 = _agg_kernel(OUT, True)(row_p, col_p, g2)
    z_pad = _tc_c(agg2[0], agg2[1], g2, dpT, b2.reshape(1, OUT))
    return z_pad[:N]


# speedup vs baseline: 1.1839x; 1.0884x over previous
import functools

import jax
import jax.numpy as jnp
from jax import lax
from jax.experimental import pallas as pl
from jax.experimental.pallas import tpu as pltpu
from jax.experimental.pallas import tpu_sc as plsc

N = 10000
NPAD = 10240
E = 320000
IN_DIM = 128
HID = 64
OUT = 32

NC = 2
NS = 16
NW = NC * NS
CH = 128
K = 80
EPT = K * CH
EPAD = EPT * NW
RPT = NPAD // NS
RReal = N // NS
Q = 8
F = 4
CHD = 40
KD = E // (NW * CHD)
DLAG = 8


def _mesh():
    return plsc.VectorSubcoreMesh(
        core_axis_name="c", subcore_axis_name="s",
        num_cores=NC, num_subcores=NS)


def _deg_kernel():
    @functools.partial(
        pl.kernel,
        out_type=jax.ShapeDtypeStruct((NC, NPAD), jnp.float32),
        mesh=_mesh(),
        compiler_params=pltpu.CompilerParams(use_tc_tiling_on_sc=False),
        scratch_types=[
            pltpu.VMEM((KD, CHD), jnp.int32),
            pltpu.VMEM((48,), jnp.float32),
            pltpu.VMEM((RPT,), jnp.float32),
            pltpu.SemaphoreType.DMA,
            pltpu.VMEM_SHARED((NPAD,), jnp.float32),
        ],
    )
    def deg(col_hbm, out_hbm, col_v, ones_v, stage_v, ssem, acc_sh):
        c = lax.axis_index("c")
        s = lax.axis_index("s")
        eb = c * NS + s
        r0 = s * RPT

        def fill_ones(i, carry):
            ones_v[pl.ds(i * 16, 16)] = jnp.full((16,), 1.0, jnp.float32)
            return carry

        lax.fori_loop(0, 3, fill_ones, 0)

        def fill_zero(i, carry):
            stage_v[pl.ds(i * 16, 16)] = jnp.zeros((16,), jnp.float32)
            return carry

        lax.fori_loop(0, RPT // 16, fill_zero, 0)
        pltpu.sync_copy(stage_v, acc_sh.at[pl.ds(r0, RPT)])
        pltpu.sync_copy(col_hbm.at[eb], col_v)
        plsc.subcore_barrier()

        def body(j, carry):
            pltpu.async_copy(
                ones_v.at[pl.ds(0, CHD)], acc_sh.at[col_v.at[j]], ssem,
                add=True)

            @pl.when(j >= DLAG)
            def _drain():
                pltpu.make_async_copy(
                    out_hbm.at[0, pl.ds(0, CHD)], ones_v.at[pl.ds(0, CHD)],
                    ssem).wait()

            return carry

        lax.fori_loop(0, KD, body, 0)
        for _ in range(DLAG):
            pltpu.make_async_copy(
                out_hbm.at[0, pl.ds(0, CHD)], ones_v.at[pl.ds(0, CHD)],
                ssem).wait()
        plsc.subcore_barrier()
        pltpu.sync_copy(acc_sh.at[pl.ds(r0, RPT)], stage_v)
        pltpu.sync_copy(stage_v, out_hbm.at[c, pl.ds(r0, RPT)])

    return deg


def _agg_kernel(D, spmem_gather):
    scratch = [
        pltpu.VMEM((K, CH), jnp.int32),
        pltpu.VMEM((K, CH), jnp.int32),
        pltpu.VMEM((Q, CH, D), jnp.float32),
        pltpu.SemaphoreType.DMA((Q,)),
        pltpu.SemaphoreType.DMA((Q,)),
        pltpu.VMEM_SHARED((NPAD, D), jnp.float32),
    ]
    if spmem_gather:
        scratch.append(pltpu.VMEM_SHARED((N, D), jnp.float32))

    @functools.partial(
        pl.kernel,
        out_type=jax.ShapeDtypeStruct((NC, N, D), jnp.float32),
        mesh=_mesh(),
        compiler_params=pltpu.CompilerParams(use_tc_tiling_on_sc=False),
        scratch_types=scratch,
    )
    def agg(row_hbm, col_hbm, g_hbm, out_hbm,
            row_v, col_v, bufs, gsem, ssem, acc_sh, *maybe_gsh):
        c = lax.axis_index("c")
        s = lax.axis_index("s")
        eb = c * NS + s
        rr = s * RReal
        g_src = maybe_gsh[0] if spmem_gather else g_hbm

        for t in range(5):
            rows = CH if t < 4 else RReal - 4 * CH
            slab = bufs.at[0, pl.ds(0, rows)]
            pltpu.sync_copy(g_hbm.at[pl.ds(rr + t * CH, rows)], slab)
            pltpu.sync_copy(slab, acc_sh.at[pl.ds(rr + t * CH, rows)])
            if spmem_gather:
                pltpu.sync_copy(slab, maybe_gsh[0].at[pl.ds(rr + t * CH, rows)])
        pltpu.sync_copy(row_hbm.at[eb], row_v)
        pltpu.sync_copy(col_hbm.at[eb], col_v)
        plsc.subcore_barrier()

        for b in range(F):
            pltpu.async_copy(g_src.at[row_v.at[b]], bufs.at[b], gsem.at[b])

        def outer(jo, carry):
            for b in range(Q):
                j = jo * Q + b
                pltpu.make_async_copy(
                    g_src.at[row_v.at[j]], bufs.at[b], gsem.at[b]).wait()
                pltpu.async_copy(
                    bufs.at[b], acc_sh.at[col_v.at[j]], ssem.at[b], add=True)
                jf = j + F
                bf = (b + F) % Q

                @pl.when(jf < K)
                def _fire():
                    @pl.when(jf >= Q)
                    def _drain():
                        pltpu.make_async_copy(
                            g_hbm.at[pl.ds(0, CH)], bufs.at[bf],
                            ssem.at[bf]).wait()

                    pltpu.async_copy(
                        g_src.at[row_v.at[jf]], bufs.at[bf], gsem.at[bf])
            return carry

        lax.fori_loop(0, K // Q, outer, 0)
        for b in range(Q):
            pltpu.make_async_copy(
                g_hbm.at[pl.ds(0, CH)], bufs.at[b], ssem.at[b]).wait()
        plsc.subcore_barrier()

        for t in range(5):
            rows = CH if t < 4 else RReal - 4 * CH
            slab = bufs.at[0, pl.ds(0, rows)]
            pltpu.sync_copy(acc_sh.at[pl.ds(rr + t * CH, rows)], slab)
            pltpu.sync_copy(slab, out_hbm.at[c, pl.ds(rr + t * CH, rows)])

    return agg


_R = 1000


def _dinv_block(dp_ref):
    deg = dp_ref[:, 0:1] + dp_ref[:, 1:2] + 1.0
    return lax.rsqrt(deg)


def _tc_a(x, W1, dpT):
    def body(x_ref, w_ref, dp_ref, o_ref):
        dinv = _dinv_block(dp_ref)
        o_ref[...] = jnp.dot(x_ref[...], w_ref[...],
                             preferred_element_type=jnp.float32) * dinv

    return pl.pallas_call(
        body,
        grid=(N // _R,),
        in_specs=[
            pl.BlockSpec((_R, IN_DIM), lambda i: (i, 0)),
            pl.BlockSpec((IN_DIM, HID), lambda i: (0, 0)),
            pl.BlockSpec((_R, NC), lambda i: (i, 0)),
        ],
        out_specs=pl.BlockSpec((_R, HID), lambda i: (i, 0)),
        out_shape=jax.ShapeDtypeStruct((N, HID), jnp.float32),
    )(x, W1, dpT)


def _tc_b(parts, g1, dpT, b1, W2):
    def body(p_ref, g1_ref, dp_ref, b_ref, w_ref, o_ref):
        dinv = _dinv_block(dp_ref)
        pre = (p_ref[0] + p_ref[1] - g1_ref[...]) * dinv + b_ref[...]
        h = jnp.maximum(pre, 0.0)
        o_ref[...] = jnp.dot(h, w_ref[...],
                             preferred_element_type=jnp.float32) * dinv

    return pl.pallas_call(
        body,
        grid=(N // _R,),
        in_specs=[
            pl.BlockSpec((NC, _R, HID), lambda i: (0, i, 0)),
            pl.BlockSpec((_R, HID), lambda i: (i, 0)),
            pl.BlockSpec((_R, NC), lambda i: (i, 0)),
            pl.BlockSpec((1, HID), lambda i: (0, 0)),
            pl.BlockSpec((HID, OUT), lambda i: (0, 0)),
        ],
        out_specs=pl.BlockSpec((_R, OUT), lambda i: (i, 0)),
        out_shape=jax.ShapeDtypeStruct((N, OUT), jnp.float32),
    )(parts, g1, dpT, b1, W2)


def _tc_c(parts, g2, dpT, b2):
    def body(p_ref, g2_ref, dp_ref, b_ref, o_ref):
        dinv = _dinv_block(dp_ref)
        o_ref[...] = (p_ref[0] + p_ref[1] - g2_ref[...]) * dinv + b_ref[...]

    return pl.pallas_call(
        body,
        grid=(N // _R,),
        in_specs=[
            pl.BlockSpec((NC, _R, OUT), lambda i: (0, i, 0)),
            pl.BlockSpec((_R, OUT), lambda i: (i, 0)),
            pl.BlockSpec((_R, NC), lambda i: (i, 0)),
            pl.BlockSpec((1, OUT), lambda i: (0, 0)),
        ],
        out_specs=pl.BlockSpec((_R, OUT), lambda i: (i, 0)),
        out_shape=jax.ShapeDtypeStruct((N, OUT), jnp.float32),
    )(parts, g2, dpT, b2)


def kernel(x, edge_index, W1, b1, W2, b2):
    row = edge_index[0].astype(jnp.int32)
    col = edge_index[1].astype(jnp.int32)
    epad = EPAD - E
    pad_i = jnp.arange(epad, dtype=jnp.int32)
    row_p = jnp.concatenate([row, (pad_i * 97) % N]).reshape(NW, K, CH)
    col_p = jnp.concatenate([col, N + pad_i % (NPAD - N)]).reshape(NW, K, CH)

    deg_part = _deg_kernel()(col.reshape(NW, KD, CHD))
    dpT = deg_part.T[:N]

    g1 = _tc_a(x, W1, dpT)
    agg1 = _agg_kernel(HID, False)(row_p, col_p, g1)
    g2 = _tc_b(agg1, g1, dpT, b1.reshape(1, HID), W2)
    agg2 = _agg_kernel(OUT, True)(row_p, col_p, g2)
    return _tc_c(agg2, g2, dpT, b2.reshape(1, OUT))


# speedup vs baseline: 1.2219x; 1.0321x over previous
import functools

import jax
import jax.numpy as jnp
from jax import lax
from jax.experimental import pallas as pl
from jax.experimental.pallas import tpu as pltpu
from jax.experimental.pallas import tpu_sc as plsc

N = 10000
NPAD = 10240
E = 320000
IN_DIM = 128
HID = 64
OUT = 32

NC = 2
NS = 16
NW = NC * NS
CH = 128
K = 80
EPT = K * CH
EPAD = EPT * NW
RPT = NPAD // NS
RReal = N // NS
Q = 8
F = 4
CHD = 40
KD = E // (NW * CHD)
DLAG = 8


def _mesh():
    return plsc.VectorSubcoreMesh(
        core_axis_name="c", subcore_axis_name="s",
        num_cores=NC, num_subcores=NS)


def _deg_kernel():
    @functools.partial(
        pl.kernel,
        out_type=jax.ShapeDtypeStruct((NC, NPAD), jnp.float32),
        mesh=_mesh(),
        compiler_params=pltpu.CompilerParams(use_tc_tiling_on_sc=False),
        scratch_types=[
            pltpu.VMEM((KD, CHD), jnp.int32),
            pltpu.VMEM((48,), jnp.float32),
            pltpu.VMEM((RPT,), jnp.float32),
            pltpu.SemaphoreType.DMA,
            pltpu.VMEM_SHARED((NPAD,), jnp.float32),
        ],
    )
    def deg(col_hbm, out_hbm, col_v, ones_v, stage_v, ssem, acc_sh):
        c = lax.axis_index("c")
        s = lax.axis_index("s")
        eb = c * NS + s
        r0 = s * RPT

        def fill_ones(i, carry):
            ones_v[pl.ds(i * 16, 16)] = jnp.full((16,), 1.0, jnp.float32)
            return carry

        lax.fori_loop(0, 3, fill_ones, 0)

        def fill_zero(i, carry):
            stage_v[pl.ds(i * 16, 16)] = jnp.zeros((16,), jnp.float32)
            return carry

        lax.fori_loop(0, RPT // 16, fill_zero, 0)
        pltpu.sync_copy(stage_v, acc_sh.at[pl.ds(r0, RPT)])
        pltpu.sync_copy(col_hbm.at[eb], col_v)
        plsc.subcore_barrier()

        def body(j, carry):
            pltpu.async_copy(
                ones_v.at[pl.ds(0, CHD)], acc_sh.at[col_v.at[j]], ssem,
                add=True)

            @pl.when(j >= DLAG)
            def _drain():
                pltpu.make_async_copy(
                    out_hbm.at[0, pl.ds(0, CHD)], ones_v.at[pl.ds(0, CHD)],
                    ssem).wait()

            return carry

        lax.fori_loop(0, KD, body, 0)
        for _ in range(DLAG):
            pltpu.make_async_copy(
                out_hbm.at[0, pl.ds(0, CHD)], ones_v.at[pl.ds(0, CHD)],
                ssem).wait()
        plsc.subcore_barrier()
        pltpu.sync_copy(acc_sh.at[pl.ds(r0, RPT)], stage_v)
        pltpu.sync_copy(stage_v, out_hbm.at[c, pl.ds(r0, RPT)])

    return deg


def _agg_kernel(D, spmem_gather):
    scratch = [
        pltpu.VMEM((K, CH), jnp.int32),
        pltpu.VMEM((K, CH), jnp.int32),
        pltpu.VMEM((Q, CH, D), jnp.float32),
        pltpu.SemaphoreType.DMA((Q,)),
        pltpu.SemaphoreType.DMA((Q,)),
        pltpu.VMEM_SHARED((NPAD, D), jnp.float32),
    ]
    if spmem_gather:
        scratch.append(pltpu.VMEM_SHARED((N, D), jnp.float32))

    @functools.partial(
        pl.kernel,
        out_type=jax.ShapeDtypeStruct((NC, N, D), jnp.float32),
        mesh=_mesh(),
        compiler_params=pltpu.CompilerParams(use_tc_tiling_on_sc=False),
        scratch_types=scratch,
    )
    def agg(row_hbm, col_hbm, g_hbm, out_hbm,
            row_v, col_v, bufs, gsem, ssem, acc_sh, *maybe_gsh):
        c = lax.axis_index("c")
        s = lax.axis_index("s")
        eb = c * NS + s
        rr = s * RReal
        g_src = maybe_gsh[0] if spmem_gather else g_hbm

        for t in range(5):
            rows = CH if t < 4 else RReal - 4 * CH
            slab = bufs.at[0, pl.ds(0, rows)]
            pltpu.sync_copy(g_hbm.at[pl.ds(rr + t * CH, rows)], slab)
            pltpu.sync_copy(slab, acc_sh.at[pl.ds(rr + t * CH, rows)])
            if spmem_gather:
                pltpu.sync_copy(slab, maybe_gsh[0].at[pl.ds(rr + t * CH, rows)])
        pltpu.sync_copy(row_hbm.at[eb], row_v)
        pltpu.sync_copy(col_hbm.at[eb], col_v)
        plsc.subcore_barrier()

        for b in range(F):
            pltpu.async_copy(g_src.at[row_v.at[b]], bufs.at[b], gsem.at[b])

        def outer(jo, carry):
            for b in range(Q):
                j = jo * Q + b
                pltpu.make_async_copy(
                    g_src.at[row_v.at[j]], bufs.at[b], gsem.at[b]).wait()
                pltpu.async_copy(
                    bufs.at[b], acc_sh.at[col_v.at[j]], ssem.at[b], add=True)
                jf = j + F
                bf = (b + F) % Q

                @pl.when(jf < K)
                def _fire():
                    @pl.when(jf >= Q)
                    def _drain():
                        pltpu.make_async_copy(
                            g_hbm.at[pl.ds(0, CH)], bufs.at[bf],
                            ssem.at[bf]).wait()

                    pltpu.async_copy(
                        g_src.at[row_v.at[jf]], bufs.at[bf], gsem.at[bf])
            return carry

        lax.fori_loop(0, K // Q, outer, 0)
        for b in range(Q):
            pltpu.make_async_copy(
                g_hbm.at[pl.ds(0, CH)], bufs.at[b], ssem.at[b]).wait()
        plsc.subcore_barrier()

        for t in range(5):
            rows = CH if t < 4 else RReal - 4 * CH
            slab = bufs.at[0, pl.ds(0, rows)]
            pltpu.sync_copy(acc_sh.at[pl.ds(rr + t * CH, rows)], slab)
            pltpu.sync_copy(slab, out_hbm.at[c, pl.ds(rr + t * CH, rows)])

    return agg


_R = 2000


def _dinv_block(dp_ref):
    deg = dp_ref[:, 0:1] + dp_ref[:, 1:2] + 1.0
    return lax.rsqrt(deg)


def _tc_a(x, W1, dpT):
    def body(x_ref, w_ref, dp_ref, o_ref):
        dinv = _dinv_block(dp_ref)
        o_ref[...] = jnp.dot(x_ref[...], w_ref[...],
                             preferred_element_type=jnp.float32) * dinv

    return pl.pallas_call(
        body,
        grid=(N // _R,),
        in_specs=[
            pl.BlockSpec((_R, IN_DIM), lambda i: (i, 0)),
            pl.BlockSpec((IN_DIM, HID), lambda i: (0, 0)),
            pl.BlockSpec((_R, NC), lambda i: (i, 0)),
        ],
        out_specs=pl.BlockSpec((_R, HID), lambda i: (i, 0)),
        out_shape=jax.ShapeDtypeStruct((N, HID), jnp.float32),
    )(x, W1, dpT)


def _tc_b(parts, g1, dpT, b1, W2):
    def body(p_ref, g1_ref, dp_ref, b_ref, w_ref, o_ref):
        dinv = _dinv_block(dp_ref)
        pre = (p_ref[0] + p_ref[1] - g1_ref[...]) * dinv + b_ref[...]
        h = jnp.maximum(pre, 0.0)
        o_ref[...] = jnp.dot(h, w_ref[...],
                             preferred_element_type=jnp.float32) * dinv

    return pl.pallas_call(
        body,
        grid=(N // _R,),
        in_specs=[
            pl.BlockSpec((NC, _R, HID), lambda i: (0, i, 0)),
            pl.BlockSpec((_R, HID), lambda i: (i, 0)),
            pl.BlockSpec((_R, NC), lambda i: (i, 0)),
            pl.BlockSpec((1, HID), lambda i: (0, 0)),
            pl.BlockSpec((HID, OUT), lambda i: (0, 0)),
        ],
        out_specs=pl.BlockSpec((_R, OUT), lambda i: (i, 0)),
        out_shape=jax.ShapeDtypeStruct((N, OUT), jnp.float32),
    )(parts, g1, dpT, b1, W2)


def _tc_c(parts, g2, dpT, b2):
    def body(p_ref, g2_ref, dp_ref, b_ref, o_ref):
        dinv = _dinv_block(dp_ref)
        o_ref[...] = (p_ref[0] + p_ref[1] - g2_ref[...]) * dinv + b_ref[...]

    return pl.pallas_call(
        body,
        grid=(N // _R,),
        in_specs=[
            pl.BlockSpec((NC, _R, OUT), lambda i: (0, i, 0)),
            pl.BlockSpec((_R, OUT), lambda i: (i, 0)),
            pl.BlockSpec((_R, NC), lambda i: (i, 0)),
            pl.BlockSpec((1, OUT), lambda i: (0, 0)),
        ],
        out_specs=pl.BlockSpec((_R, OUT), lambda i: (i, 0)),
        out_shape=jax.ShapeDtypeStruct((N, OUT), jnp.float32),
    )(parts, g2, dpT, b2)


def kernel(x, edge_index, W1, b1, W2, b2):
    row = edge_index[0].astype(jnp.int32)
    col = edge_index[1].astype(jnp.int32)
    epad = EPAD - E
    col_raw = col.reshape(NW, KD, CHD)
    row_b, col_b = lax.optimization_barrier((row, col))
    pad_i = jnp.arange(epad, dtype=jnp.int32)
    row_p = jnp.concatenate([row_b, (pad_i * 97) % N]).reshape(NW, K, CH)
    col_p = jnp.concatenate([col_b, N + pad_i % (NPAD - N)]).reshape(NW, K, CH)

    deg_part = _deg_kernel()(col_raw)
    dpT = deg_part.T[:N]

    g1 = _tc_a(x, W1, dpT)
    agg1 = _agg_kernel(HID, False)(row_p, col_p, g1)
    g2 = _tc_b(agg1, g1, dpT, b1.reshape(1, HID), W2)
    agg2 = _agg_kernel(OUT, True)(row_p, col_p, g2)
    return _tc_c(agg2, g2, dpT, b2.reshape(1, OUT))


# speedup vs baseline: 1.2428x; 1.0171x over previous
import functools

import jax
import jax.numpy as jnp
from jax import lax
from jax.experimental import pallas as pl
from jax.experimental.pallas import tpu as pltpu
from jax.experimental.pallas import tpu_sc as plsc

N = 10000
NPAD = 10240
E = 320000
IN_DIM = 128
HID = 64
OUT = 32

NC = 2
NS = 16
NW = NC * NS
CH = 128
K = 80
EPT = K * CH
EPAD = EPT * NW
RPT = NPAD // NS
RReal = N // NS
CHD = 80
KD = E // (NW * CHD)
DLAG = 8


def _mesh():
    return plsc.VectorSubcoreMesh(
        core_axis_name="c", subcore_axis_name="s",
        num_cores=NC, num_subcores=NS)


def _deg_kernel():
    @functools.partial(
        pl.kernel,
        out_type=jax.ShapeDtypeStruct((NC, NPAD), jnp.float32),
        mesh=_mesh(),
        compiler_params=pltpu.CompilerParams(use_tc_tiling_on_sc=False),
        scratch_types=[
            pltpu.VMEM((KD, CHD), jnp.int32),
            pltpu.VMEM((CHD,), jnp.float32),
            pltpu.VMEM((RPT,), jnp.float32),
            pltpu.SemaphoreType.DMA,
            pltpu.VMEM_SHARED((NPAD,), jnp.float32),
        ],
    )
    def deg(col_hbm, out_hbm, col_v, ones_v, stage_v, ssem, acc_sh):
        c = lax.axis_index("c")
        s = lax.axis_index("s")
        eb = c * NS + s
        r0 = s * RPT

        def fill_ones(i, carry):
            ones_v[pl.ds(i * 16, 16)] = jnp.full((16,), 1.0, jnp.float32)
            return carry

        lax.fori_loop(0, CHD // 16, fill_ones, 0)

        def fill_zero(i, carry):
            stage_v[pl.ds(i * 16, 16)] = jnp.zeros((16,), jnp.float32)
            return carry

        lax.fori_loop(0, RPT // 16, fill_zero, 0)
        pltpu.sync_copy(stage_v, acc_sh.at[pl.ds(r0, RPT)])
        pltpu.sync_copy(col_hbm.at[eb], col_v)
        plsc.subcore_barrier()

        def body(j, carry):
            pltpu.async_copy(
                ones_v.at[pl.ds(0, CHD)], acc_sh.at[col_v.at[j]], ssem,
                add=True)

            @pl.when(j >= DLAG)
            def _drain():
                pltpu.make_async_copy(
                    out_hbm.at[0, pl.ds(0, CHD)], ones_v.at[pl.ds(0, CHD)],
                    ssem).wait()

            return carry

        lax.fori_loop(0, KD, body, 0)
        for _ in range(DLAG):
            pltpu.make_async_copy(
                out_hbm.at[0, pl.ds(0, CHD)], ones_v.at[pl.ds(0, CHD)],
                ssem).wait()
        plsc.subcore_barrier()
        pltpu.sync_copy(acc_sh.at[pl.ds(r0, RPT)], stage_v)
        pltpu.sync_copy(stage_v, out_hbm.at[c, pl.ds(r0, RPT)])

    return deg


def _agg_kernel(D, spmem_gather, Q, F):
    scratch = [
        pltpu.VMEM((K, CH), jnp.int32),
        pltpu.VMEM((K, CH), jnp.int32),
        pltpu.VMEM((Q, CH, D), jnp.float32),
        pltpu.SemaphoreType.DMA((Q,)),
        pltpu.SemaphoreType.DMA((Q,)),
        pltpu.VMEM_SHARED((NPAD, D), jnp.float32),
    ]
    if spmem_gather:
        scratch.append(pltpu.VMEM_SHARED((N, D), jnp.float32))

    @functools.partial(
        pl.kernel,
        out_type=jax.ShapeDtypeStruct((NC, N, D), jnp.float32),
        mesh=_mesh(),
        compiler_params=pltpu.CompilerParams(use_tc_tiling_on_sc=False),
        scratch_types=scratch,
    )
    def agg(row_hbm, col_hbm, g_hbm, out_hbm,
            row_v, col_v, bufs, gsem, ssem, acc_sh, *maybe_gsh):
        c = lax.axis_index("c")
        s = lax.axis_index("s")
        eb = c * NS + s
        rr = s * RReal
        g_src = maybe_gsh[0] if spmem_gather else g_hbm

        for t in range(5):
            rows = CH if t < 4 else RReal - 4 * CH
            slab = bufs.at[0, pl.ds(0, rows)]
            pltpu.sync_copy(g_hbm.at[pl.ds(rr + t * CH, rows)], slab)
            pltpu.sync_copy(slab, acc_sh.at[pl.ds(rr + t * CH, rows)])
            if spmem_gather:
                pltpu.sync_copy(slab, maybe_gsh[0].at[pl.ds(rr + t * CH, rows)])
        pltpu.sync_copy(row_hbm.at[eb], row_v)
        pltpu.sync_copy(col_hbm.at[eb], col_v)
        plsc.subcore_barrier()

        for b in range(F):
            pltpu.async_copy(g_src.at[row_v.at[b]], bufs.at[b], gsem.at[b])

        def outer(jo, carry):
            for b in range(Q):
                j = jo * Q + b
                pltpu.make_async_copy(
                    g_src.at[row_v.at[j]], bufs.at[b], gsem.at[b]).wait()
                pltpu.async_copy(
                    bufs.at[b], acc_sh.at[col_v.at[j]], ssem.at[b], add=True)
                jf = j + F
                bf = (b + F) % Q

                @pl.when(jf < K)
                def _fire():
                    @pl.when(jf >= Q)
                    def _drain():
                        pltpu.make_async_copy(
                            g_hbm.at[pl.ds(0, CH)], bufs.at[bf],
                            ssem.at[bf]).wait()

                    pltpu.async_copy(
                        g_src.at[row_v.at[jf]], bufs.at[bf], gsem.at[bf])
            return carry

        lax.fori_loop(0, K // Q, outer, 0)
        for b in range(Q):
            pltpu.make_async_copy(
                g_hbm.at[pl.ds(0, CH)], bufs.at[b], ssem.at[b]).wait()
        plsc.subcore_barrier()

        for t in range(5):
            rows = CH if t < 4 else RReal - 4 * CH
            slab = bufs.at[0, pl.ds(0, rows)]
            pltpu.sync_copy(acc_sh.at[pl.ds(rr + t * CH, rows)], slab)
            pltpu.sync_copy(slab, out_hbm.at[c, pl.ds(rr + t * CH, rows)])

    return agg


_R = 2000


def _dinv_block(dp_ref):
    deg = dp_ref[:, 0:1] + dp_ref[:, 1:2] + 1.0
    return lax.rsqrt(deg)


def _tc_a(x, W1, dpT):
    def body(x_ref, w_ref, dp_ref, o_ref):
        dinv = _dinv_block(dp_ref)
        o_ref[...] = jnp.dot(x_ref[...], w_ref[...],
                             preferred_element_type=jnp.float32) * dinv

    return pl.pallas_call(
        body,
        grid=(N // _R,),
        in_specs=[
            pl.BlockSpec((_R, IN_DIM), lambda i: (i, 0)),
            pl.BlockSpec((IN_DIM, HID), lambda i: (0, 0)),
            pl.BlockSpec((_R, NC), lambda i: (i, 0)),
        ],
        out_specs=pl.BlockSpec((_R, HID), lambda i: (i, 0)),
        out_shape=jax.ShapeDtypeStruct((N, HID), jnp.float32),
    )(x, W1, dpT)


def _tc_b(parts, g1, dpT, b1, W2):
    def body(p_ref, g1_ref, dp_ref, b_ref, w_ref, o_ref):
        dinv = _dinv_block(dp_ref)
        pre = (p_ref[0] + p_ref[1] - g1_ref[...]) * dinv + b_ref[...]
        h = jnp.maximum(pre, 0.0)
        o_ref[...] = jnp.dot(h, w_ref[...],
                             preferred_element_type=jnp.float32) * dinv

    return pl.pallas_call(
        body,
        grid=(N // _R,),
        in_specs=[
            pl.BlockSpec((NC, _R, HID), lambda i: (0, i, 0)),
            pl.BlockSpec((_R, HID), lambda i: (i, 0)),
            pl.BlockSpec((_R, NC), lambda i: (i, 0)),
            pl.BlockSpec((1, HID), lambda i: (0, 0)),
            pl.BlockSpec((HID, OUT), lambda i: (0, 0)),
        ],
        out_specs=pl.BlockSpec((_R, OUT), lambda i: (i, 0)),
        out_shape=jax.ShapeDtypeStruct((N, OUT), jnp.float32),
    )(parts, g1, dpT, b1, W2)


def _tc_c(parts, g2, dpT, b2):
    def body(p_ref, g2_ref, dp_ref, b_ref, o_ref):
        dinv = _dinv_block(dp_ref)
        o_ref[...] = (p_ref[0] + p_ref[1] - g2_ref[...]) * dinv + b_ref[...]

    return pl.pallas_call(
        body,
        grid=(N // _R,),
        in_specs=[
            pl.BlockSpec((NC, _R, OUT), lambda i: (0, i, 0)),
            pl.BlockSpec((_R, OUT), lambda i: (i, 0)),
            pl.BlockSpec((_R, NC), lambda i: (i, 0)),
            pl.BlockSpec((1, OUT), lambda i: (0, 0)),
        ],
        out_specs=pl.BlockSpec((_R, OUT), lambda i: (i, 0)),
        out_shape=jax.ShapeDtypeStruct((N, OUT), jnp.float32),
    )(parts, g2, dpT, b2)


def kernel(x, edge_index, W1, b1, W2, b2):
    row = edge_index[0].astype(jnp.int32)
    col = edge_index[1].astype(jnp.int32)
    epad = EPAD - E
    col_raw = col.reshape(NW, KD, CHD)
    row_b, col_b = lax.optimization_barrier((row, col))
    pad_i = jnp.arange(epad, dtype=jnp.int32)
    row_p = jnp.concatenate([row_b, (pad_i * 97) % N]).reshape(NW, K, CH)
    col_p = jnp.concatenate([col_b, N + pad_i % (NPAD - N)]).reshape(NW, K, CH)

    deg_part = _deg_kernel()(col_raw)
    dpT = deg_part.T[:N]

    g1 = _tc_a(x, W1, dpT)
    agg1 = _agg_kernel(HID, False, 8, 5)(row_p, col_p, g1)
    g2 = _tc_b(agg1, g1, dpT, b1.reshape(1, HID), W2)
    agg2 = _agg_kernel(OUT, True, 8, 5)(row_p, col_p, g2)
    return _tc_c(agg2, g2, dpT, b2.reshape(1, OUT))


# speedup vs baseline: 1.2606x; 1.0144x over previous
import functools

import jax
import jax.numpy as jnp
from jax import lax
from jax.experimental import pallas as pl
from jax.experimental.pallas import tpu as pltpu
from jax.experimental.pallas import tpu_sc as plsc

N = 10000
NPAD = 10240
E = 320000
IN_DIM = 128
HID = 64
OUT = 32

NC = 2
NS = 16
NW = NC * NS
CH = 128
K = 80
EPT = K * CH
EPAD = EPT * NW
RPT = NPAD // NS
RReal = N // NS
CHD = 80
KD = E // (NW * CHD)
DLAG = 8


def _mesh():
    return plsc.VectorSubcoreMesh(
        core_axis_name="c", subcore_axis_name="s",
        num_cores=NC, num_subcores=NS)


def _deg_kernel():
    @functools.partial(
        pl.kernel,
        out_type=jax.ShapeDtypeStruct((NC, NPAD), jnp.float32),
        mesh=_mesh(),
        compiler_params=pltpu.CompilerParams(use_tc_tiling_on_sc=False),
        scratch_types=[
            pltpu.VMEM((KD, CHD), jnp.int32),
            pltpu.VMEM((CHD,), jnp.float32),
            pltpu.VMEM((RPT,), jnp.float32),
            pltpu.SemaphoreType.DMA,
            pltpu.VMEM_SHARED((NPAD,), jnp.float32),
        ],
    )
    def deg(col_hbm, out_hbm, col_v, ones_v, stage_v, ssem, acc_sh):
        c = lax.axis_index("c")
        s = lax.axis_index("s")
        eb = c * NS + s
        r0 = s * RPT

        def fill_ones(i, carry):
            ones_v[pl.ds(i * 16, 16)] = jnp.full((16,), 1.0, jnp.float32)
            return carry

        lax.fori_loop(0, CHD // 16, fill_ones, 0)

        def fill_zero(i, carry):
            stage_v[pl.ds(i * 16, 16)] = jnp.zeros((16,), jnp.float32)
            return carry

        lax.fori_loop(0, RPT // 16, fill_zero, 0)
        pltpu.sync_copy(stage_v, acc_sh.at[pl.ds(r0, RPT)])
        pltpu.sync_copy(col_hbm.at[eb], col_v)
        plsc.subcore_barrier()

        def body(j, carry):
            pltpu.async_copy(
                ones_v.at[pl.ds(0, CHD)], acc_sh.at[col_v.at[j]], ssem,
                add=True)

            @pl.when(j >= DLAG)
            def _drain():
                pltpu.make_async_copy(
                    out_hbm.at[0, pl.ds(0, CHD)], ones_v.at[pl.ds(0, CHD)],
                    ssem).wait()

            return carry

        lax.fori_loop(0, KD, body, 0)
        for _ in range(DLAG):
            pltpu.make_async_copy(
                out_hbm.at[0, pl.ds(0, CHD)], ones_v.at[pl.ds(0, CHD)],
                ssem).wait()
        plsc.subcore_barrier()
        pltpu.sync_copy(acc_sh.at[pl.ds(r0, RPT)], stage_v)
        pltpu.sync_copy(stage_v, out_hbm.at[c, pl.ds(r0, RPT)])

    return deg


def _agg_kernel(D, spmem_gather, Q, F):
    scratch = [
        pltpu.VMEM((K, CH), jnp.int32),
        pltpu.VMEM((K, CH), jnp.int32),
        pltpu.VMEM((Q, CH, D), jnp.float32),
        pltpu.SemaphoreType.DMA((Q,)),
        pltpu.SemaphoreType.DMA((Q,)),
        pltpu.VMEM_SHARED((NPAD, D), jnp.float32),
    ]
    if spmem_gather:
        scratch.append(pltpu.VMEM_SHARED((N, D), jnp.float32))

    @functools.partial(
        pl.kernel,
        out_type=jax.ShapeDtypeStruct((NC, N, D), jnp.float32),
        mesh=_mesh(),
        compiler_params=pltpu.CompilerParams(use_tc_tiling_on_sc=False),
        scratch_types=scratch,
    )
    def agg(row_hbm, col_hbm, g_hbm, out_hbm,
            row_v, col_v, bufs, gsem, ssem, acc_sh, *maybe_gsh):
        c = lax.axis_index("c")
        s = lax.axis_index("s")
        eb = c * NS + s
        rr = s * RReal
        g_src = maybe_gsh[0] if spmem_gather else g_hbm

        for t in range(5):
            rows = CH if t < 4 else RReal - 4 * CH
            slab = bufs.at[0, pl.ds(0, rows)]
            pltpu.sync_copy(g_hbm.at[pl.ds(rr + t * CH, rows)], slab)
            pltpu.sync_copy(slab, acc_sh.at[pl.ds(rr + t * CH, rows)])
            if spmem_gather:
                pltpu.sync_copy(slab, maybe_gsh[0].at[pl.ds(rr + t * CH, rows)])
        pltpu.sync_copy(row_hbm.at[eb], row_v)
        pltpu.sync_copy(col_hbm.at[eb], col_v)
        plsc.subcore_barrier()

        for b in range(F):
            pltpu.async_copy(g_src.at[row_v.at[b]], bufs.at[b], gsem.at[b])

        def outer(jo, carry):
            for b in range(Q):
                j = jo * Q + b
                pltpu.make_async_copy(
                    g_src.at[row_v.at[j]], bufs.at[b], gsem.at[b]).wait()
                pltpu.async_copy(
                    bufs.at[b], acc_sh.at[col_v.at[j]], ssem.at[b], add=True)
                jf = j + F
                bf = (b + F) % Q

                @pl.when(jf < K)
                def _fire():
                    @pl.when(jf >= Q)
                    def _drain():
                        pltpu.make_async_copy(
                            g_hbm.at[pl.ds(0, CH)], bufs.at[bf],
                            ssem.at[bf]).wait()

                    pltpu.async_copy(
                        g_src.at[row_v.at[jf]], bufs.at[bf], gsem.at[bf])
            return carry

        lax.fori_loop(0, K // Q, outer, 0)
        for b in range(Q):
            pltpu.make_async_copy(
                g_hbm.at[pl.ds(0, CH)], bufs.at[b], ssem.at[b]).wait()
        plsc.subcore_barrier()

        for t in range(5):
            rows = CH if t < 4 else RReal - 4 * CH
            slab = bufs.at[0, pl.ds(0, rows)]
            pltpu.sync_copy(acc_sh.at[pl.ds(rr + t * CH, rows)], slab)
            pltpu.sync_copy(slab, out_hbm.at[c, pl.ds(rr + t * CH, rows)])

    return agg


_R = 2000


def _dinv_block(dp_ref):
    deg = dp_ref[:, 0:1] + dp_ref[:, 1:2] + 1.0
    return lax.rsqrt(deg)


def _tc_a(x, W1, dpT):
    def body(x_ref, w_ref, dp_ref, o_ref):
        dinv = _dinv_block(dp_ref)
        o_ref[...] = jnp.dot(x_ref[...], w_ref[...],
                             preferred_element_type=jnp.float32) * dinv

    return pl.pallas_call(
        body,
        grid=(N // _R,),
        in_specs=[
            pl.BlockSpec((_R, IN_DIM), lambda i: (i, 0)),
            pl.BlockSpec((IN_DIM, HID), lambda i: (0, 0)),
            pl.BlockSpec((_R, NC), lambda i: (i, 0)),
        ],
        out_specs=pl.BlockSpec((_R, HID), lambda i: (i, 0)),
        out_shape=jax.ShapeDtypeStruct((N, HID), jnp.float32),
    )(x, W1, dpT)


def _tc_b(parts, g1, dpT, b1, W2):
    def body(p_ref, g1_ref, dp_ref, b_ref, w_ref, o_ref):
        dinv = _dinv_block(dp_ref)
        pre = (p_ref[0] + p_ref[1] - g1_ref[...]) * dinv + b_ref[...]
        h = jnp.maximum(pre, 0.0)
        o_ref[...] = jnp.dot(h, w_ref[...],
                             preferred_element_type=jnp.float32) * dinv

    return pl.pallas_call(
        body,
        grid=(N // _R,),
        in_specs=[
            pl.BlockSpec((NC, _R, HID), lambda i: (0, i, 0)),
            pl.BlockSpec((_R, HID), lambda i: (i, 0)),
            pl.BlockSpec((_R, NC), lambda i: (i, 0)),
            pl.BlockSpec((1, HID), lambda i: (0, 0)),
            pl.BlockSpec((HID, OUT), lambda i: (0, 0)),
        ],
        out_specs=pl.BlockSpec((_R, OUT), lambda i: (i, 0)),
        out_shape=jax.ShapeDtypeStruct((N, OUT), jnp.float32),
    )(parts, g1, dpT, b1, W2)


def _tc_c(parts, g2, dpT, b2):
    def body(p_ref, g2_ref, dp_ref, b_ref, o_ref):
        dinv = _dinv_block(dp_ref)
        o_ref[...] = (p_ref[0] + p_ref[1] - g2_ref[...]) * dinv + b_ref[...]

    return pl.pallas_call(
        body,
        grid=(N // _R,),
        in_specs=[
            pl.BlockSpec((NC, _R, OUT), lambda i: (0, i, 0)),
            pl.BlockSpec((_R, OUT), lambda i: (i, 0)),
            pl.BlockSpec((_R, NC), lambda i: (i, 0)),
            pl.BlockSpec((1, OUT), lambda i: (0, 0)),
        ],
        out_specs=pl.BlockSpec((_R, OUT), lambda i: (i, 0)),
        out_shape=jax.ShapeDtypeStruct((N, OUT), jnp.float32),
    )(parts, g2, dpT, b2)


def kernel(x, edge_index, W1, b1, W2, b2):
    row = edge_index[0].astype(jnp.int32)
    col = edge_index[1].astype(jnp.int32)
    epad = EPAD - E
    col_raw = col.reshape(NW, KD, CHD)
    row_b, col_b = lax.optimization_barrier((row, col))
    pad_i = jnp.arange(epad, dtype=jnp.int32)
    row_p = jnp.concatenate([row_b, (pad_i * 97) % N]).reshape(NW, K, CH)
    col_p = jnp.concatenate([col_b, N + pad_i % (NPAD - N)]).reshape(NW, K, CH)

    deg_part = _deg_kernel()(col_raw)
    dpT = deg_part.T[:N]

    g1 = _tc_a(x, W1, dpT)
    agg1 = _agg_kernel(HID, False, 8, 6)(row_p, col_p, g1)
    g2 = _tc_b(agg1, g1, dpT, b1.reshape(1, HID), W2)
    agg2 = _agg_kernel(OUT, True, 8, 5)(row_p, col_p, g2)
    return _tc_c(agg2, g2, dpT, b2.reshape(1, OUT))
